# baseline (speedup 1.0000x reference)
import jax
import jax.numpy as jnp
from jax import lax
from jax.experimental import pallas as pl
from jax.experimental.pallas import tpu as pltpu
from jax.experimental.pallas import tpu_sc as plsc

SIGMA = 1.0
LR = 0.1
NUM = 1024
N = 512
NC = 2
NS = 16
NW = NC * NS
NSC = NUM // 2
RPT = NSC // NW
L = 16
NCH = N // L
BLK = 128

_mesh = plsc.VectorSubcoreMesh(
    core_axis_name="c", subcore_axis_name="s", num_cores=NC, num_subcores=NS)


def _lanemin(v):
    iot = lax.iota(jnp.int32, L)
    for k in (8, 4, 2, 1):
        v = jnp.minimum(v, v.at[iot ^ k].get(mode="promise_in_bounds"))
    return v


def _dists_body(x_hbm, w_hbm, mins_hbm, x_v, w_v, red_v, acc_v, sem0):
    tid = lax.axis_index("c") * NS + lax.axis_index("s")
    base = tid * RPT
    cp0 = pltpu.async_copy(w_hbm.at[pl.ds(base, RPT)], w_v, sem0)
    pltpu.sync_copy(x_hbm, x_v)
    xc = [x_v[pl.ds(j * L, L)] for j in range(NCH)]
    iot = lax.iota(jnp.int32, L)

    def row_step(i, _):
        accs = [jnp.zeros((L,), jnp.float32) for _ in range(4)]
        for j in range(NCH):
            d = w_v[i, pl.ds(j * L, L)] - xc[j]
            accs[j % 4] = accs[j % 4] + d * d
        acc_v[pl.ds(i * L, L)] = (accs[0] + accs[1]) + (accs[2] + accs[3])
        return 0

    cp0.wait()
    lax.fori_loop(0, RPT, row_step, 0)
    cols = [plsc.load_gather(acc_v, [iot * L + j]) for j in range(L)]
    for step in (8, 4, 2, 1):
        cols = [cols[t] + cols[t + step] for t in range(step)]
    dists16 = cols[0]

    gminv = _lanemin(dists16)
    cand = jnp.where(dists16 == gminv,
                     (base + iot).astype(jnp.float32), jnp.float32(2e9))
    gidxv = _lanemin(cand)
    red_v[:] = jnp.where(iot == 0, gminv, jnp.where(iot == 1, gidxv, 0.0))
    pltpu.sync_copy(red_v, mins_hbm.at[tid])


_dists_call = pl.kernel(
    _dists_body,
    out_type=jax.ShapeDtypeStruct((NW, L), jnp.float32),
    mesh=_mesh,
    compiler_params=pltpu.CompilerParams(needs_layout_passes=False),
    scratch_types=[
        pltpu.VMEM((N,), jnp.float32),
        pltpu.VMEM((RPT, N), jnp.float32),
        pltpu.VMEM((L,), jnp.float32),
        pltpu.VMEM((L * L,), jnp.float32),
        pltpu.SemaphoreType.DMA,
    ],
)


def _tcdists_body(x_ref, w_ref, out_ref):
    d = w_ref[...] - x_ref[...]
    out_ref[...] = jnp.sum(d * d, axis=1, keepdims=True)


_tcdists_call = pl.pallas_call(
    _tcdists_body,
    grid=(NSC // BLK,),
    in_specs=[
        pl.BlockSpec((1, N), lambda i: (0, 0)),
        pl.BlockSpec((BLK, N), lambda i: (i + NSC // BLK, 0)),
    ],
    out_specs=pl.BlockSpec((BLK, 1), lambda i: (i, 0)),
    out_shape=jax.ShapeDtypeStruct((NSC, 1), jnp.float32),
)


def _upd_body(mins_ref, d2_ref, nhb_hbm, x_ref, w_ref, out_ref,
              latt_row, latt_col, sem):
    i = pl.program_id(0)

    @pl.when(i == 0)
    def _():
        vals = mins_ref[:, 0:1]
        idxs = mins_ref[:, 1:2]
        m1 = jnp.min(vals)
        i1 = jnp.min(jnp.where(vals == m1, idxs, jnp.float32(2e9)))
        v2 = d2_ref[...]
        m2 = jnp.min(v2)
        rows2 = lax.broadcasted_iota(jnp.int32, (NSC, 1), 0).astype(jnp.float32) + NSC
        i2 = jnp.min(jnp.where(v2 == m2, rows2, jnp.float32(2e9)))
        bidx = jnp.where(m1 <= m2, i1, i2)
        bmu = bidx.astype(jnp.int32)
        cp = pltpu.make_async_copy(nhb_hbm.at[bmu], latt_row, sem)
        cp.start()
        cp.wait()
        lr = jnp.exp(latt_row[...] * -0.5)
        eye = (lax.broadcasted_iota(jnp.int32, (BLK, BLK), 0) ==
               lax.broadcasted_iota(jnp.int32, (BLK, BLK), 1)
               ).astype(jnp.float32)
        for k in range(NUM // BLK):
            seg = lr[k * BLK:(k + 1) * BLK].reshape(1, BLK)
            latt_col[pl.ds(k * BLK, BLK), :] = lax.dot_general(
                eye, seg, (((1,), (1,)), ((), ())),
                precision=lax.Precision.HIGHEST)

    w = w_ref[...]
    lc = latt_col[pl.ds(i * BLK, BLK), :]
    out_ref[...] = w + (LR * lc) * (x_ref[...] - w)


_upd_call = pl.pallas_call(
    _upd_body,
    grid=(NUM // BLK,),
    in_specs=[
        pl.BlockSpec((NW, L), lambda i: (0, 0)),
        pl.BlockSpec((NSC, 1), lambda i: (0, 0)),
        pl.BlockSpec(memory_space=pl.ANY),
        pl.BlockSpec((1, N), lambda i: (0, 0)),
        pl.BlockSpec((BLK, N), lambda i: (i, 0)),
    ],
    out_specs=pl.BlockSpec((BLK, N), lambda i: (i, 0)),
    out_shape=jax.ShapeDtypeStruct((NUM, N), jnp.float32),
    scratch_shapes=[
        pltpu.VMEM((NUM,), jnp.float32),
        pltpu.VMEM((NUM, 1), jnp.float32),
        pltpu.SemaphoreType.DMA,
    ],
)


@jax.jit
def kernel(x, W, nhb_dists):
    x2 = x.reshape(1, N)
    mins = _dists_call(x.reshape(N), W)
    d2 = _tcdists_call(x2, W)
    return _upd_call(mins, d2, nhb_dists, x2, W)
